# baseline (speedup 1.0000x reference)
import functools

import jax
import jax.numpy as jnp
from jax import lax
from jax.experimental import pallas as pl
from jax.experimental.pallas import tpu as pltpu
from jax.experimental.pallas import tpu_sc as plsc

MAX_LENGTH = 200
DIM = 64
BATCH = 4096
SEQ = 200

_NC, _NS = 2, 16
_NW = _NC * _NS
_NSLAB = 4
_SLAB = BATCH // _NSLAB
_RPW = _SLAB // _NW
_BUF_B = 4
_NCOPY = _RPW // _BUF_B


def _sc_body(off, tab_hbm, out_hbm, row_v, buf, sem):
    wid = lax.axis_index("s") * _NC + lax.axis_index("c")
    base = wid * _RPW

    pltpu.sync_copy(tab_hbm.at[pl.ds(MAX_LENGTH - off, 1)], row_v)
    chunks = [row_v[0, pl.ds(16 * j, 16)] for j in range(DIM // 16)]

    def fill(s, carry):
        for b in range(_BUF_B):
            for j in range(DIM // 16):
                buf[b, pl.ds(s * DIM + 16 * j, 16)] = chunks[j]
        return carry

    lax.fori_loop(0, SEQ, fill, 0)

    copies = [
        pltpu.async_copy(buf, out_hbm.at[pl.ds(base + i * _BUF_B, _BUF_B)], sem)
        for i in range(_NCOPY)
    ]
    for c in copies:
        c.wait()


def _make_slab_kernel(off):
    return pl.kernel(
        functools.partial(_sc_body, off),
        mesh=plsc.VectorSubcoreMesh(core_axis_name="c", subcore_axis_name="s"),
        out_type=jax.ShapeDtypeStruct((_SLAB, SEQ * DIM), jnp.float32),
        scratch_types=[
            pltpu.VMEM((1, DIM), jnp.float32),
            pltpu.VMEM((_BUF_B, SEQ * DIM), jnp.float32),
            pltpu.SemaphoreType.DMA,
        ],
    )


def kernel(inputs, kernel):
    del inputs
    parts = [_make_slab_kernel(i)(kernel[i:]) for i in range(_NSLAB)]
    return jnp.concatenate(
        [jnp.reshape(p, (_SLAB, SEQ, DIM)) for p in parts], axis=0)


# speedup vs baseline: 1.0954x; 1.0954x over previous
import jax
import jax.numpy as jnp
from jax import lax
from jax.experimental import pallas as pl
from jax.experimental.pallas import tpu as pltpu
from jax.experimental.pallas import tpu_sc as plsc

MAX_LENGTH = 200
DIM = 64
BATCH = 4096
SEQ = 200

_TCB = 2816
_BB = 256
_NCOPY_TC = _TCB // _BB

_SCB = BATCH - _TCB
_NC, _NS = 2, 16
_NW = _NC * _NS
_RPW = _SCB // _NW
_BUF_B = 4
_NCOPY_SC = _RPW // _BUF_B


def _tc_kernel(tab_ref, out_ref, rowbuf, scratch, sems):
    row = tab_ref[MAX_LENGTH, :]
    for s in range(SEQ):
        rowbuf[:, pl.ds(s * DIM, DIM)] = row[None, :]
    scratch[...] = jnp.broadcast_to(rowbuf[...], scratch.shape)
    for i in range(_NCOPY_TC):
        pltpu.make_async_copy(
            scratch, out_ref.at[pl.ds(i * _BB, _BB)], sems.at[i]).start()
    for i in range(_NCOPY_TC):
        pltpu.make_async_copy(
            scratch, out_ref.at[pl.ds(i * _BB, _BB)], sems.at[i]).wait()


def _sc_body(tab_hbm, out_hbm, row_v, buf, sem):
    wid = lax.axis_index("s") * _NC + lax.axis_index("c")
    base = wid * _RPW
    pltpu.sync_copy(tab_hbm.at[pl.ds(MAX_LENGTH, 1)], row_v)
    chunks = [row_v[0, pl.ds(16 * j, 16)] for j in range(DIM // 16)]

    def fill(s, carry):
        for b in range(_BUF_B):
            for j in range(DIM // 16):
                buf[b, pl.ds(s * DIM + 16 * j, 16)] = chunks[j]
        return carry

    lax.fori_loop(0, SEQ, fill, 0)

    copies = [
        pltpu.async_copy(buf, out_hbm.at[pl.ds(base + i * _BUF_B, _BUF_B)], sem)
        for i in range(_NCOPY_SC)
    ]
    for c in copies:
        c.wait()


def kernel(inputs, kernel):
    del inputs
    flat_tc = pl.pallas_call(
        _tc_kernel,
        in_specs=[pl.BlockSpec(memory_space=pltpu.MemorySpace.VMEM)],
        out_specs=pl.BlockSpec(memory_space=pltpu.MemorySpace.HBM),
        out_shape=jax.ShapeDtypeStruct((_TCB, SEQ * DIM), jnp.float32),
        scratch_shapes=[
            pltpu.VMEM((1, SEQ * DIM), jnp.float32),
            pltpu.VMEM((_BB, SEQ * DIM), jnp.float32),
            pltpu.SemaphoreType.DMA((_NCOPY_TC,)),
        ],
    )(kernel)

    flat_sc = pl.kernel(
        _sc_body,
        mesh=plsc.VectorSubcoreMesh(core_axis_name="c", subcore_axis_name="s"),
        out_type=jax.ShapeDtypeStruct((_SCB, SEQ * DIM), jnp.float32),
        scratch_types=[
            pltpu.VMEM((1, DIM), jnp.float32),
            pltpu.VMEM((_BUF_B, SEQ * DIM), jnp.float32),
            pltpu.SemaphoreType.DMA,
        ],
    )(kernel)

    flat = jnp.concatenate([flat_tc, flat_sc], axis=0)
    return jnp.reshape(flat, (BATCH, SEQ, DIM))


# speedup vs baseline: 1.5542x; 1.4188x over previous
import jax
import jax.numpy as jnp
from jax.experimental import pallas as pl
from jax.experimental.pallas import tpu as pltpu

MAX_LENGTH = 200
DIM = 64
BATCH = 4096
SEQ = 200

_BB = 512
_NCOPY = BATCH // _BB


def _fanout_kernel(tab_ref, out_ref, rowbuf, scratch, sems):
    row = tab_ref[MAX_LENGTH, :]
    for s in range(SEQ):
        rowbuf[:, pl.ds(s * DIM, DIM)] = row[None, :]
    scratch[...] = jnp.broadcast_to(rowbuf[...], scratch.shape)
    for i in range(_NCOPY):
        pltpu.make_async_copy(
            scratch, out_ref.at[pl.ds(i * _BB, _BB)], sems.at[i]).start()
    for i in range(_NCOPY):
        pltpu.make_async_copy(
            scratch, out_ref.at[pl.ds(i * _BB, _BB)], sems.at[i]).wait()


def kernel(inputs, kernel):
    del inputs
    flat = pl.pallas_call(
        _fanout_kernel,
        in_specs=[pl.BlockSpec(memory_space=pltpu.MemorySpace.VMEM)],
        out_specs=pl.BlockSpec(memory_space=pltpu.MemorySpace.HBM),
        out_shape=jax.ShapeDtypeStruct((BATCH, SEQ * DIM), jnp.float32),
        scratch_shapes=[
            pltpu.VMEM((1, SEQ * DIM), jnp.float32),
            pltpu.VMEM((_BB, SEQ * DIM), jnp.float32),
            pltpu.SemaphoreType.DMA((_NCOPY,)),
        ],
    )(kernel)
    return jnp.reshape(flat, (BATCH, SEQ, DIM))


# speedup vs baseline: 1.5803x; 1.0168x over previous
import jax
import jax.numpy as jnp
from jax.experimental import pallas as pl
from jax.experimental.pallas import tpu as pltpu

MAX_LENGTH = 200
DIM = 64
BATCH = 4096
SEQ = 200

_BB = 256
_NCOPY = BATCH // _BB


def _fanout_kernel(tab_ref, out_ref, rowbuf, scratch, sems):
    row = tab_ref[MAX_LENGTH, :]
    for s in range(SEQ):
        rowbuf[:, pl.ds(s * DIM, DIM)] = row[None, :]
    scratch[...] = jnp.broadcast_to(rowbuf[...], scratch.shape)
    for i in range(_NCOPY):
        pltpu.make_async_copy(
            scratch, out_ref.at[pl.ds(i * _BB, _BB)], sems.at[i]).start()
    for i in range(_NCOPY):
        pltpu.make_async_copy(
            scratch, out_ref.at[pl.ds(i * _BB, _BB)], sems.at[i]).wait()


def kernel(inputs, kernel):
    del inputs
    flat = pl.pallas_call(
        _fanout_kernel,
        in_specs=[pl.BlockSpec(memory_space=pltpu.MemorySpace.VMEM)],
        out_specs=pl.BlockSpec(memory_space=pltpu.MemorySpace.HBM),
        out_shape=jax.ShapeDtypeStruct((BATCH, SEQ * DIM), jnp.float32),
        scratch_shapes=[
            pltpu.VMEM((1, SEQ * DIM), jnp.float32),
            pltpu.VMEM((_BB, SEQ * DIM), jnp.float32),
            pltpu.SemaphoreType.DMA((_NCOPY,)),
        ],
    )(kernel)
    return jnp.reshape(flat, (BATCH, SEQ, DIM))


# speedup vs baseline: 1.5852x; 1.0031x over previous
import jax
import jax.numpy as jnp
from jax.experimental import pallas as pl
from jax.experimental.pallas import tpu as pltpu

MAX_LENGTH = 200
DIM = 64
BATCH = 4096
SEQ = 200

_BB = 128
_NCOPY = BATCH // _BB


def _fanout_kernel(tab_ref, out_ref, rowbuf, scratch, sems):
    row = tab_ref[MAX_LENGTH, :]
    for s in range(SEQ):
        rowbuf[:, pl.ds(s * DIM, DIM)] = row[None, :]
    scratch[...] = jnp.broadcast_to(rowbuf[...], scratch.shape)
    for i in range(_NCOPY):
        pltpu.make_async_copy(
            scratch, out_ref.at[pl.ds(i * _BB, _BB)], sems.at[i]).start()
    for i in range(_NCOPY):
        pltpu.make_async_copy(
            scratch, out_ref.at[pl.ds(i * _BB, _BB)], sems.at[i]).wait()


def kernel(inputs, kernel):
    del inputs
    flat = pl.pallas_call(
        _fanout_kernel,
        in_specs=[pl.BlockSpec(memory_space=pltpu.MemorySpace.VMEM)],
        out_specs=pl.BlockSpec(memory_space=pltpu.MemorySpace.HBM),
        out_shape=jax.ShapeDtypeStruct((BATCH, SEQ * DIM), jnp.float32),
        scratch_shapes=[
            pltpu.VMEM((1, SEQ * DIM), jnp.float32),
            pltpu.VMEM((_BB, SEQ * DIM), jnp.float32),
            pltpu.SemaphoreType.DMA((_NCOPY,)),
        ],
    )(kernel)
    return jnp.reshape(flat, (BATCH, SEQ, DIM))
